# baseline (speedup 1.0000x reference)
import functools

import jax
import jax.numpy as jnp
from jax import lax
from jax.experimental import pallas as pl
from jax.experimental.pallas import tpu as pltpu
from jax.experimental.pallas import tpu_sc as plsc

_dot = functools.partial(
    jnp.dot, preferred_element_type=jnp.float32,
    precision=jax.lax.Precision.HIGHEST)

N = 10000
E = 320000
DN = 128
DE = 16
H = 256
G = 128
NG = 64

N_PAD = 10240
NB = 2048
EB = 8000
EC = 6400
NCH = E // EC
GPW = 2
EPW = E // 32
ECS = 2000
NSCH = EPW // ECS


def _sc_hist_body(row_hbm, col_hbm, batch_hbm, ea_hbm, out_hbm, se_hbm,
                  table_v, batch_v, rb0, cb0, rb1, cb1,
                  se_v, eb0, eb1, kb0, kb1, sem0, sem1, sem2, sem3):
    nc = 2
    wid = lax.axis_index("s") * nc + lax.axis_index("c")

    z = jnp.zeros((16,), jnp.float32)

    def _zero(i, _):
        table_v[0, pl.ds(i * 16, 16)] = z
        table_v[1, pl.ds(i * 16, 16)] = z
        return 0

    lax.fori_loop(0, N_PAD // 16, _zero, 0, unroll=8)

    pltpu.sync_copy(batch_hbm, batch_v)

    ones = jnp.ones((16,), jnp.float32)
    sebase = wid * EPW
    iota16 = lax.iota(jnp.int32, 16)

    def _se_start(cs, eb, kb, sem):
        off = sebase + cs * ECS
        pltpu.async_copy(ea_hbm.at[pl.ds(off * DE, ECS * DE)], eb, sem)
        pltpu.async_copy(col_hbm.at[pl.ds(off, ECS)], kb, sem)

    def _se_wait(cs, eb, kb, sem):
        off = sebase + cs * ECS
        pltpu.make_async_copy(ea_hbm.at[pl.ds(off * DE, ECS * DE)], eb, sem).wait()
        pltpu.make_async_copy(col_hbm.at[pl.ds(off, ECS)], kb, sem).wait()

    _se_start(0, eb0, kb0, sem2)
    _se_start(1, eb1, kb1, sem3)

    def _start(c, rb, cb, sem):
        pltpu.async_copy(row_hbm.at[pl.ds(c * EC, EC)], rb, sem)
        pltpu.async_copy(col_hbm.at[pl.ds(c * EC, EC)], cb, sem)

    def _wait(c, rb, cb, sem):
        pltpu.make_async_copy(row_hbm.at[pl.ds(c * EC, EC)], rb, sem).wait()
        pltpu.make_async_copy(col_hbm.at[pl.ds(c * EC, EC)], cb, sem).wait()

    def _process(rb, cb):
        @plsc.parallel_loop(0, EC // 16, unroll=8)
        def _step(j):
            colv = cb[pl.ds(j * 16, 16)]
            rowv = rb[pl.ds(j * 16, 16)]
            gev = plsc.load_gather(batch_v, [colv])
            m = lax.shift_right_logical(gev, 1) == wid
            sub = lax.bitwise_and(gev, 1)
            plsc.addupdate_scatter(table_v, [sub, rowv], ones, mask=m)

    _start(0, rb0, cb0, sem0)

    def _chunk_pair(c2, _):
        c = c2 * 2
        _start(c + 1, rb1, cb1, sem1)
        _wait(c, rb0, cb0, sem0)
        _process(rb0, cb0)

        @pl.when(c + 2 < NCH)
        def _():
            _start(c + 2, rb0, cb0, sem0)

        _wait(c + 1, rb1, cb1, sem1)
        _process(rb1, cb1)
        return 0

    lax.fori_loop(0, NCH // 2, _chunk_pair, 0)

    pltpu.sync_copy(table_v, out_hbm.at[pl.ds(wid * GPW, GPW)])

    def _zero_se(i, _):
        se_v[pl.ds(i * 16, 16)] = jnp.zeros((16,), jnp.float32)
        return 0

    lax.fori_loop(0, NG * DE // 16, _zero_se, 0, unroll=8)

    def _se_process(eb, kb):
        @plsc.parallel_loop(0, ECS // 16, unroll=5)
        def _sj(j):
            colv = kb[pl.ds(j * 16, 16)]
            gev = plsc.load_gather(batch_v, [colv])
            gbase = gev * DE
            rvec = (iota16 + j * 16) * DE
            for d in range(DE):
                vals = plsc.load_gather(eb, [rvec + d])
                plsc.addupdate_scatter(se_v, [gbase + d], vals)

    def _se_chunk_pair(c2, _):
        cs = c2 * 2
        _se_wait(cs, eb0, kb0, sem2)
        _se_process(eb0, kb0)

        @pl.when(cs + 2 < NSCH)
        def _():
            _se_start(cs + 2, eb0, kb0, sem2)

        _se_wait(cs + 1, eb1, kb1, sem3)
        _se_process(eb1, kb1)

        @pl.when(cs + 3 < NSCH)
        def _():
            _se_start(cs + 3, eb1, kb1, sem3)

        return 0

    lax.fori_loop(0, NSCH // 2, _se_chunk_pair, 0)

    if NSCH % 2 == 1:
        _se_wait(NSCH - 1, eb0, kb0, sem2)
        _se_process(eb0, kb0)

    pltpu.sync_copy(se_v, se_hbm.at[wid])


def _sc_hist(row, col, batch, ea_flat):
    mesh = plsc.VectorSubcoreMesh(core_axis_name="c", subcore_axis_name="s")
    f = functools.partial(
        pl.kernel,
        mesh=mesh,
        compiler_params=pltpu.CompilerParams(needs_layout_passes=False),
        out_type=[
            jax.ShapeDtypeStruct((NG, N_PAD), jnp.float32),
            jax.ShapeDtypeStruct((32, NG * DE), jnp.float32),
        ],
        scratch_types=[
            pltpu.VMEM((GPW, N_PAD), jnp.float32),
            pltpu.VMEM((N,), jnp.int32),
            pltpu.VMEM((EC,), jnp.int32),
            pltpu.VMEM((EC,), jnp.int32),
            pltpu.VMEM((EC,), jnp.int32),
            pltpu.VMEM((EC,), jnp.int32),
            pltpu.VMEM((NG * DE,), jnp.float32),
            pltpu.VMEM((ECS * DE,), jnp.float32),
            pltpu.VMEM((ECS * DE,), jnp.float32),
            pltpu.VMEM((ECS,), jnp.int32),
            pltpu.VMEM((ECS,), jnp.int32),
            pltpu.SemaphoreType.DMA,
            pltpu.SemaphoreType.DMA,
            pltpu.SemaphoreType.DMA,
            pltpu.SemaphoreType.DMA,
        ],
    )(_sc_hist_body)
    return f(row, col, batch, ea_flat)


def _tc_nodes_body(x_ref, b_ref, np_ref):
    i = pl.program_id(0)

    @pl.when(i == 0)
    def _():
        np_ref[...] = jnp.zeros_like(np_ref)

    xb = x_ref[...]
    bb = b_ref[0, 0, :]
    gi = lax.broadcasted_iota(jnp.int32, (NG, NB), 0)
    oh = (gi == bb[None, :]).astype(jnp.float32)
    np_ref[...] += _dot(oh, xb)


def _tc_nodes(x_pad, batch3):
    return pl.pallas_call(
        _tc_nodes_body,
        grid=(N_PAD // NB,),
        in_specs=[
            pl.BlockSpec((NB, DN), lambda i: (i, 0)),
            pl.BlockSpec((1, 1, NB), lambda i: (i, 0, 0)),
        ],
        out_specs=pl.BlockSpec((NG, DN), lambda i: (0, 0)),
        out_shape=jax.ShapeDtypeStruct((NG, DN), jnp.float32),
    )(x_pad, batch3)


def _tc_sx_body(c_ref, x_ref, sx_ref, cnt_ref):
    i = pl.program_id(0)

    @pl.when(i == 0)
    def _():
        sx_ref[...] = jnp.zeros_like(sx_ref)
        cnt_ref[...] = jnp.zeros_like(cnt_ref)

    cb = c_ref[...]
    sx_ref[...] += _dot(cb, x_ref[...])
    cnt_ref[...] += jnp.sum(cb, axis=1)[:, None]


def _tc_sx(C, x_pad):
    return pl.pallas_call(
        _tc_sx_body,
        grid=(N_PAD // NB,),
        in_specs=[
            pl.BlockSpec((NG, NB), lambda i: (0, i)),
            pl.BlockSpec((NB, DN), lambda i: (i, 0)),
        ],
        out_specs=[
            pl.BlockSpec((NG, DN), lambda i: (0, 0)),
            pl.BlockSpec((NG, G), lambda i: (0, 0)),
        ],
        out_shape=[
            jax.ShapeDtypeStruct((NG, DN), jnp.float32),
            jax.ShapeDtypeStruct((NG, G), jnp.float32),
        ],
    )(C, x_pad)


def _tc_head_body(sx_ref, np_ref, sep_ref, cnt_ref,
                  w10, b10, w11, b11, w12, b12,
                  w20, b20, w21, b21, w22, b22, out_ref):
    dot = _dot
    t = dot(w11[...], w12[...])
    a = dot(w10[...], t)
    c1 = dot(b10[...], t) + dot(b11[...], w12[...]) + b12[...]
    cnt1 = jnp.max(cnt_ref[...], axis=1, keepdims=True)
    se = sep_ref[0]
    for k in range(1, 32):
        se = se + sep_ref[k]
    ep = (dot(sx_ref[...], a[:DN])
          + dot(se, a[DN:DN + DE])
          + cnt1 * c1)
    h = jnp.concatenate([np_ref[...], ep], axis=1)
    h = dot(h, w20[...]) + b20[...]
    h = dot(h, w21[...]) + b21[...]
    out_ref[...] = dot(h, w22[...]) + b22[...]


def _tc_head(Sx, NP, se_parts, cnt, W1_0, b1_0, W1_1, b1_1, W1_2, b1_2,
             W2_0, b2_0, W2_1, b2_1, W2_2, b2_2):
    args = (Sx, NP, se_parts, cnt,
            W1_0, b1_0.reshape(1, H), W1_1, b1_1.reshape(1, H),
            W1_2, b1_2.reshape(1, H),
            W2_0, b2_0.reshape(1, H), W2_1, b2_1.reshape(1, H),
            W2_2, b2_2.reshape(1, G))
    return pl.pallas_call(
        _tc_head_body,
        out_shape=jax.ShapeDtypeStruct((NG, G), jnp.float32),
    )(*args)


def kernel(x, edge_index, edge_attr, u, batch,
           W1_0, b1_0, W1_1, b1_1, W1_2, b1_2,
           W2_0, b2_0, W2_1, b2_1, W2_2, b2_2):
    row = edge_index[0]
    col = edge_index[1]
    x_pad = jnp.concatenate(
        [x, jnp.zeros((N_PAD - N, DN), jnp.float32)], axis=0)
    batch_pad = jnp.concatenate(
        [batch, jnp.full((N_PAD - N,), NG, jnp.int32)])
    batch3 = batch_pad.reshape(N_PAD // NB, 1, NB)

    C, se_parts = _sc_hist(row, col, batch, edge_attr.reshape(E * DE))
    se_parts = se_parts.reshape(32, NG, DE)
    NP = _tc_nodes(x_pad, batch3)
    Sx, cnt = _tc_sx(C, x_pad)
    return _tc_head(Sx, NP, se_parts, cnt,
                    W1_0, b1_0, W1_1, b1_1, W1_2, b1_2,
                    W2_0, b2_0, W2_1, b2_1, W2_2, b2_2)


# speedup vs baseline: 1.3917x; 1.3917x over previous
import functools

import jax
import jax.numpy as jnp
from jax import lax
from jax.experimental import pallas as pl
from jax.experimental.pallas import tpu as pltpu
from jax.experimental.pallas import tpu_sc as plsc

_dot = functools.partial(
    jnp.dot, preferred_element_type=jnp.float32,
    precision=jax.lax.Precision.HIGHEST)

N = 10000
E = 320000
DN = 128
DE = 16
H = 256
G = 128
NG = 64

N_PAD = 10240
NB = 2048
EB = 8000
EC = 6400
NCH = E // EC
GPW = 2


def _sc_hist_body(row_hbm, col_hbm, batch_hbm, out_hbm,
                  table_v, batch_v, rb0, cb0, rb1, cb1, sem0, sem1):
    nc = 2
    wid = lax.axis_index("s") * nc + lax.axis_index("c")

    z = jnp.zeros((16,), jnp.float32)

    def _zero(i, _):
        table_v[0, pl.ds(i * 16, 16)] = z
        table_v[1, pl.ds(i * 16, 16)] = z
        return 0

    lax.fori_loop(0, N_PAD // 16, _zero, 0, unroll=8)

    pltpu.sync_copy(batch_hbm, batch_v)

    ones = jnp.ones((16,), jnp.float32)

    def _start(c, rb, cb, sem):
        pltpu.async_copy(row_hbm.at[pl.ds(c * EC, EC)], rb, sem)
        pltpu.async_copy(col_hbm.at[pl.ds(c * EC, EC)], cb, sem)

    def _wait(c, rb, cb, sem):
        pltpu.make_async_copy(row_hbm.at[pl.ds(c * EC, EC)], rb, sem).wait()
        pltpu.make_async_copy(col_hbm.at[pl.ds(c * EC, EC)], cb, sem).wait()

    def _process(rb, cb):
        @plsc.parallel_loop(0, EC // 16, unroll=8)
        def _step(j):
            colv = cb[pl.ds(j * 16, 16)]
            rowv = rb[pl.ds(j * 16, 16)]
            gev = plsc.load_gather(batch_v, [colv])
            m = lax.shift_right_logical(gev, 1) == wid
            sub = lax.bitwise_and(gev, 1)
            plsc.addupdate_scatter(table_v, [sub, rowv], ones, mask=m)

    _start(0, rb0, cb0, sem0)

    def _chunk_pair(c2, _):
        c = c2 * 2
        _start(c + 1, rb1, cb1, sem1)
        _wait(c, rb0, cb0, sem0)
        _process(rb0, cb0)

        @pl.when(c + 2 < NCH)
        def _():
            _start(c + 2, rb0, cb0, sem0)

        _wait(c + 1, rb1, cb1, sem1)
        _process(rb1, cb1)
        return 0

    lax.fori_loop(0, NCH // 2, _chunk_pair, 0)

    pltpu.sync_copy(table_v, out_hbm.at[pl.ds(wid * GPW, GPW)])


def _sc_hist(row, col, batch):
    mesh = plsc.VectorSubcoreMesh(core_axis_name="c", subcore_axis_name="s")
    f = functools.partial(
        pl.kernel,
        mesh=mesh,
        compiler_params=pltpu.CompilerParams(needs_layout_passes=False),
        out_type=jax.ShapeDtypeStruct((NG, N_PAD), jnp.float32),
        scratch_types=[
            pltpu.VMEM((GPW, N_PAD), jnp.float32),
            pltpu.VMEM((N,), jnp.int32),
            pltpu.VMEM((EC,), jnp.int32),
            pltpu.VMEM((EC,), jnp.int32),
            pltpu.VMEM((EC,), jnp.int32),
            pltpu.VMEM((EC,), jnp.int32),
            pltpu.SemaphoreType.DMA,
            pltpu.SemaphoreType.DMA,
        ],
    )(_sc_hist_body)
    return f(row, col, batch)


def _tc_pools_body(c_ref, x_ref, b_ref, sx_ref, np_ref, st_ref, cnt_ref):
    i = pl.program_id(0)

    @pl.when(i == 0)
    def _():
        sx_ref[...] = jnp.zeros_like(sx_ref)
        np_ref[...] = jnp.zeros_like(np_ref)
        st_ref[...] = jnp.zeros_like(st_ref)
        cnt_ref[...] = jnp.zeros_like(cnt_ref)

    cb = c_ref[...]
    xb = x_ref[...]
    bb = b_ref[0, 0, :]
    sx_ref[...] += _dot(cb, xb)
    cnt_ref[...] += jnp.sum(cb, axis=1)[:, None]
    gi = lax.broadcasted_iota(jnp.int32, (NG, NB), 0)
    oh = (gi == bb[None, :]).astype(jnp.float32)
    np_ref[...] += _dot(oh, xb)
    ohlt = (gi > bb[None, :]).astype(jnp.float32)
    st_ref[...] += jnp.sum(ohlt, axis=1)[:, None]


def _tc_pools(C, x_pad, batch3):
    return pl.pallas_call(
        _tc_pools_body,
        grid=(N_PAD // NB,),
        in_specs=[
            pl.BlockSpec((NG, NB), lambda i: (0, i)),
            pl.BlockSpec((NB, DN), lambda i: (i, 0)),
            pl.BlockSpec((1, 1, NB), lambda i: (i, 0, 0)),
        ],
        out_specs=[
            pl.BlockSpec((NG, DN), lambda i: (0, 0)),
            pl.BlockSpec((NG, DN), lambda i: (0, 0)),
            pl.BlockSpec((NG, G), lambda i: (0, 0)),
            pl.BlockSpec((NG, G), lambda i: (0, 0)),
        ],
        out_shape=[
            jax.ShapeDtypeStruct((NG, DN), jnp.float32),
            jax.ShapeDtypeStruct((NG, DN), jnp.float32),
            jax.ShapeDtypeStruct((NG, G), jnp.float32),
            jax.ShapeDtypeStruct((NG, G), jnp.float32),
        ],
    )(C, x_pad, batch3)


def _tc_edges_body(col_ref, ea_ref, st_ref, se_ref):
    i = pl.program_id(0)

    @pl.when(i == 0)
    def _():
        se_ref[...] = jnp.zeros_like(se_ref)

    colf = col_ref[0, 0, :].astype(jnp.float32)
    start1 = jnp.max(st_ref[...], axis=1, keepdims=True)
    cmp = (colf[None, :] >= start1).astype(jnp.float32)
    gef = jnp.sum(cmp, axis=0) - 1.0
    gi = lax.broadcasted_iota(jnp.int32, (NG, EB), 0).astype(jnp.float32)
    oh = (gi == gef[None, :]).astype(jnp.bfloat16)
    se_ref[...] += jnp.dot(oh, ea_ref[...].astype(jnp.bfloat16),
                           preferred_element_type=jnp.float32)


def _tc_edges(col3, ea, start):
    return pl.pallas_call(
        _tc_edges_body,
        grid=(E // EB,),
        in_specs=[
            pl.BlockSpec((1, 1, EB), lambda i: (i, 0, 0)),
            pl.BlockSpec((EB, DE), lambda i: (i, 0)),
            pl.BlockSpec((NG, G), lambda i: (0, 0)),
        ],
        out_specs=[
            pl.BlockSpec((NG, DE), lambda i: (0, 0)),
        ],
        out_shape=[
            jax.ShapeDtypeStruct((NG, DE), jnp.float32),
        ],
    )(col3, ea, start)


def _tc_head_body(sx_ref, np_ref, se_ref, cnt_ref,
                  w10, b10, w11, b11, w12, b12,
                  w20, b20, w21, b21, w22, b22, out_ref):
    dot = _dot
    t = dot(w11[...], w12[...])
    a = dot(w10[...], t)
    c1 = dot(b10[...], t) + dot(b11[...], w12[...]) + b12[...]
    cnt1 = jnp.max(cnt_ref[...], axis=1, keepdims=True)
    ep = (dot(sx_ref[...], a[:DN])
          + dot(se_ref[...], a[DN:DN + DE])
          + cnt1 * c1)
    h = jnp.concatenate([np_ref[...], ep], axis=1)
    h = dot(h, w20[...]) + b20[...]
    h = dot(h, w21[...]) + b21[...]
    out_ref[...] = dot(h, w22[...]) + b22[...]


def _tc_head(Sx, NP, Se, cnt, W1_0, b1_0, W1_1, b1_1, W1_2, b1_2,
             W2_0, b2_0, W2_1, b2_1, W2_2, b2_2):
    args = (Sx, NP, Se, cnt,
            W1_0, b1_0.reshape(1, H), W1_1, b1_1.reshape(1, H),
            W1_2, b1_2.reshape(1, H),
            W2_0, b2_0.reshape(1, H), W2_1, b2_1.reshape(1, H),
            W2_2, b2_2.reshape(1, G))
    return pl.pallas_call(
        _tc_head_body,
        out_shape=jax.ShapeDtypeStruct((NG, G), jnp.float32),
    )(*args)


def kernel(x, edge_index, edge_attr, u, batch,
           W1_0, b1_0, W1_1, b1_1, W1_2, b1_2,
           W2_0, b2_0, W2_1, b2_1, W2_2, b2_2):
    row = edge_index[0]
    col = edge_index[1]
    x_pad = jnp.concatenate(
        [x, jnp.zeros((N_PAD - N, DN), jnp.float32)], axis=0)
    batch_pad = jnp.concatenate(
        [batch, jnp.full((N_PAD - N,), NG, jnp.int32)])
    batch3 = batch_pad.reshape(N_PAD // NB, 1, NB)
    col3 = col.reshape(E // EB, 1, EB)

    C = _sc_hist(row, col, batch)
    Sx, NP, start, cnt = _tc_pools(C, x_pad, batch3)
    (Se,) = _tc_edges(col3, edge_attr, start)
    return _tc_head(Sx, NP, Se, cnt,
                    W1_0, b1_0, W1_1, b1_1, W1_2, b1_2,
                    W2_0, b2_0, W2_1, b2_1, W2_2, b2_2)
